# baseline (speedup 1.0000x reference)
import functools

import jax
import jax.numpy as jnp
from jax import lax
from jax.experimental import pallas as pl
from jax.experimental.pallas import tpu as pltpu
from jax.experimental.pallas import tpu_sc as plsc

N = 10000
E = 320000
P = 160000
D_IN = 128
H = 128
D_OUT = 64

NC = 2
NS = 16
NW = NC * NS
L = 16

N_PAD = 10240
ROWS_PT = N_PAD // NS
ECH = 80
EW_PAD = ECH * 128
PCH = 40
PW_PAD = PCH * 128
SC_NBUF = 8
SC_LOOK = 6
DC_NBUF = 4
DC_LOOK = 3

_mesh = plsc.VectorSubcoreMesh(core_axis_name="c", subcore_axis_name="s")
_sc_params = pltpu.CompilerParams(use_tc_tiling_on_sc=False)
_sc_params_nl = pltpu.CompilerParams(use_tc_tiling_on_sc=False,
                                     needs_layout_passes=False)


def _wid():
    return lax.axis_index("c") * NS + lax.axis_index("s")


@functools.partial(
    pl.kernel,
    out_type=(jax.ShapeDtypeStruct((N_PAD,), jnp.float32),
              jax.ShapeDtypeStruct((N_PAD,), jnp.float32)),
    mesh=_mesh,
    scratch_types=[
        pltpu.VMEM((ECH, 128), jnp.int32),
        pltpu.VMEM((128,), jnp.float32),
        pltpu.VMEM_SHARED((N_PAD,), jnp.float32),
    ],
    compiler_params=_sc_params,
)
def _deg_kernel(dst_hbm, zeros_hbm, out0, out1, dstv, onesv, acc):
    c = lax.axis_index("c")
    s = lax.axis_index("s")
    wid = _wid()
    base = s * ROWS_PT
    pltpu.sync_copy(zeros_hbm.at[pl.ds(base, ROWS_PT)],
                    acc.at[pl.ds(base, ROWS_PT)])
    pltpu.sync_copy(dst_hbm.at[wid], dstv)
    ones16 = jnp.ones((L,), jnp.float32)
    for i in range(128 // L):
        onesv[pl.ds(i * L, L)] = ones16
    plsc.subcore_barrier()

    def body(j, carry):
        pltpu.sync_copy(onesv, acc.at[dstv.at[j]], add=True)
        return carry

    lax.fori_loop(0, ECH, body, 0)
    plsc.subcore_barrier()

    @pl.when(c == 0)
    def _():
        pltpu.sync_copy(acc.at[pl.ds(base, ROWS_PT)],
                        out0.at[pl.ds(base, ROWS_PT)])

    @pl.when(c == 1)
    def _():
        pltpu.sync_copy(acc.at[pl.ds(base, ROWS_PT)],
                        out1.at[pl.ds(base, ROWS_PT)])


@functools.partial(
    pl.kernel,
    out_type=(jax.ShapeDtypeStruct((N_PAD, D_OUT), jnp.float32),
              jax.ShapeDtypeStruct((N_PAD, D_OUT), jnp.float32)),
    mesh=_mesh,
    scratch_types=(
        [pltpu.VMEM((ECH, 128), jnp.int32),
         pltpu.VMEM((ECH, 128), jnp.int32)]
        + [pltpu.VMEM((128, D_OUT), jnp.float32)] * SC_NBUF
        + [pltpu.VMEM_SHARED((N_PAD, D_OUT), jnp.float32)]
        + [pltpu.SemaphoreType.DMA] * (2 * SC_NBUF)
    ),
    compiler_params=_sc_params,
)
def _scatter_kernel(y_hbm, src_hbm, dst_hbm, zeros_hbm, out0, out1,
                    *rest):
    srcv, dstv = rest[0], rest[1]
    rowsv = list(rest[2:2 + SC_NBUF])
    acc = rest[2 + SC_NBUF]
    sems = rest[3 + SC_NBUF:]
    sem_g = sems[:SC_NBUF]
    sem_s = sems[SC_NBUF:]
    c = lax.axis_index("c")
    s = lax.axis_index("s")
    wid = _wid()
    base = s * ROWS_PT
    pltpu.sync_copy(src_hbm.at[wid], srcv)
    pltpu.sync_copy(dst_hbm.at[wid], dstv)
    for t in range(SC_LOOK):
        pltpu.async_copy(y_hbm.at[srcv.at[t]], rowsv[t % SC_NBUF],
                         sem_g[t % SC_NBUF])
    pltpu.sync_copy(zeros_hbm.at[pl.ds(base, ROWS_PT)],
                    acc.at[pl.ds(base, ROWS_PT)])
    plsc.subcore_barrier()

    def outer(jo, carry):
        t0 = jo * SC_NBUF
        for u in range(SC_NBUF):
            t = t0 + u
            bn = (u + SC_LOOK) % SC_NBUF
            tn = t + SC_LOOK
            pltpu.make_async_copy(y_hbm.at[srcv.at[u]], rowsv[u],
                                  sem_g[u]).wait()

            @pl.when(tn >= SC_NBUF)
            def _():
                pltpu.make_async_copy(rowsv[bn], acc.at[dstv.at[u]],
                                      sem_s[bn]).wait()

            @pl.when(tn < ECH)
            def _():
                pltpu.async_copy(y_hbm.at[srcv.at[tn]], rowsv[bn],
                                 sem_g[bn])

            pltpu.async_copy(rowsv[u], acc.at[dstv.at[t]], sem_s[u],
                             add=True)
        return carry

    lax.fori_loop(0, ECH // SC_NBUF, outer, 0)
    for t in range(ECH - (SC_NBUF - SC_LOOK), ECH):
        b = t % SC_NBUF
        pltpu.make_async_copy(rowsv[b], acc.at[dstv.at[b]], sem_s[b]).wait()
    plsc.subcore_barrier()

    @pl.when(c == 0)
    def _():
        pltpu.sync_copy(acc.at[pl.ds(base, ROWS_PT)],
                        out0.at[pl.ds(base, ROWS_PT)])

    @pl.when(c == 1)
    def _():
        pltpu.sync_copy(acc.at[pl.ds(base, ROWS_PT)],
                        out1.at[pl.ds(base, ROWS_PT)])


@functools.partial(
    pl.kernel,
    out_type=(jax.ShapeDtypeStruct((NW * PW_PAD,), jnp.float32),
              jax.ShapeDtypeStruct((NW * PW_PAD,), jnp.float32)),
    mesh=_mesh,
    scratch_types=(
        [pltpu.VMEM((PCH, 128), jnp.int32),
         pltpu.VMEM((PCH, 128), jnp.int32)]
        + [pltpu.VMEM((128, D_OUT), jnp.float32)] * (2 * DC_NBUF)
        + [pltpu.VMEM((PW_PAD,), jnp.float32)]
        + [pltpu.VMEM_SHARED((N_PAD, D_OUT), jnp.float32)]
        + [pltpu.SemaphoreType.DMA] * (2 * DC_NBUF)
    ),
    compiler_params=_sc_params_nl,
)
def _decode_kernel(z_hbm, pa_hbm, pb_hbm, na_hbm, nb_hbm, outp, outn,
                   *rest):
    av, bv = rest[0], rest[1]
    za = list(rest[2:2 + DC_NBUF])
    zb = list(rest[2 + DC_NBUF:2 + 2 * DC_NBUF])
    scv = rest[2 + 2 * DC_NBUF]
    zs = rest[3 + 2 * DC_NBUF]
    sems = rest[4 + 2 * DC_NBUF:]
    sem_a = sems[:DC_NBUF]
    sem_b = sems[DC_NBUF:]
    wid = _wid()
    iota = lax.iota(jnp.int32, L)
    base = lax.axis_index("s") * ROWS_PT
    pltpu.sync_copy(z_hbm.at[pl.ds(base, ROWS_PT)],
                    zs.at[pl.ds(base, ROWS_PT)])
    plsc.subcore_barrier()

    def run(a_hbm, b_hbm, out_hbm):
        pltpu.sync_copy(a_hbm.at[wid], av)
        pltpu.sync_copy(b_hbm.at[wid], bv)
        for t in range(DC_LOOK):
            b = t % DC_NBUF
            pltpu.async_copy(zs.at[av.at[t]], za[b], sem_a[b])
            pltpu.async_copy(zs.at[bv.at[t]], zb[b], sem_b[b])

        def outer(jo, carry):
            t0 = jo * DC_NBUF
            for u in range(DC_NBUF):
                t = t0 + u
                bn = (u + DC_LOOK) % DC_NBUF
                tn = t + DC_LOOK
                pltpu.make_async_copy(zs.at[av.at[u]], za[u],
                                      sem_a[u]).wait()
                pltpu.make_async_copy(zs.at[bv.at[u]], zb[u],
                                      sem_b[u]).wait()

                @pl.when(tn < PCH)
                def _():
                    pltpu.async_copy(zs.at[av.at[tn]], za[bn], sem_a[bn])
                    pltpu.async_copy(zs.at[bv.at[tn]], zb[bn], sem_b[bn])

                def group(g, carry2):
                    rows = g * L + iota
                    accs = [jnp.zeros((L,), jnp.float32) for _ in range(2)]
                    for dcol in range(D_OUT):
                        col = jnp.full((L,), dcol, jnp.int32)
                        accs[dcol % 2] = accs[dcol % 2] + (
                            plsc.load_gather(za[u], [rows, col]) *
                            plsc.load_gather(zb[u], [rows, col]))
                    acc = accs[0] + accs[1]
                    scv[pl.ds(t * 128 + g * L, L)] = acc
                    return carry2

                lax.fori_loop(0, 128 // L, group, 0)
            return carry

        lax.fori_loop(0, PCH // DC_NBUF, outer, 0)
        pltpu.sync_copy(scv, out_hbm.at[pl.ds(wid * PW_PAD, PW_PAD)])

    run(pa_hbm, pb_hbm, outp)
    run(na_hbm, nb_hbm, outn)


def _enc1_body(x_ref, w1_ref, d0_ref, d1_ref, ya_ref, yb_ref, dinv_ref):
    deg = d0_ref[...] + d1_ref[...] + 1.0
    dinv = lax.rsqrt(deg)
    dinv_ref[...] = dinv
    xw = jnp.dot(x_ref[...], w1_ref[...], preferred_element_type=jnp.float32)
    y = xw * dinv
    ya_ref[...] = y[:, :D_OUT]
    yb_ref[...] = y[:, D_OUT:]


_enc1 = pl.pallas_call(
    _enc1_body,
    out_shape=(jax.ShapeDtypeStruct((N_PAD, D_OUT), jnp.float32),
               jax.ShapeDtypeStruct((N_PAD, D_OUT), jnp.float32),
               jax.ShapeDtypeStruct((N_PAD, 1), jnp.float32)),
)


def _enc2_body(pa0_ref, pa1_ref, pb0_ref, pb1_ref, ya_ref, yb_ref,
               dinv_ref, b1_ref, w2_ref, y2_ref):
    ha = pa0_ref[...] + pa1_ref[...] + ya_ref[...]
    hb = pb0_ref[...] + pb1_ref[...] + yb_ref[...]
    h = jnp.concatenate([ha, hb], axis=1) * dinv_ref[...] + b1_ref[...]
    valid = (lax.broadcasted_iota(jnp.int32, (N_PAD, 1), 0) < N)
    z1 = jnp.where(valid, jnp.maximum(h, 0.0), 0.0)
    xw2 = jnp.dot(z1, w2_ref[...], preferred_element_type=jnp.float32)
    y2_ref[...] = xw2 * dinv_ref[...]


_enc2 = pl.pallas_call(
    _enc2_body,
    out_shape=jax.ShapeDtypeStruct((N_PAD, D_OUT), jnp.float32),
)


def _dec_body(q0_ref, q1_ref, y2_ref, dinv_ref, b2_ref, z_ref):
    z = (q0_ref[...] + q1_ref[...] + y2_ref[...]) * dinv_ref[...] + b2_ref[...]
    valid = (lax.broadcasted_iota(jnp.int32, (N_PAD, 1), 0) < N)
    z_ref[...] = jnp.where(valid, z, 0.0)


_dec = pl.pallas_call(
    _dec_body,
    out_shape=jax.ShapeDtypeStruct((N_PAD, D_OUT), jnp.float32),
)


def _prep_idx(row, nch, fill):
    pad = NW * nch * 128 - row.shape[0]
    arr = jnp.concatenate([row, jnp.full((pad,), fill, jnp.int32)])
    return arr.reshape(NW, nch, 128)


def kernel(x, edge_index, pos_edge_index, neg_edge_index, W1, b1, W2, b2):
    x_p = jnp.pad(x, ((0, N_PAD - N), (0, 0)))
    src = _prep_idx(edge_index[0], ECH, N_PAD - 1)
    dst = _prep_idx(edge_index[1], ECH, N_PAD - 1)
    pa = _prep_idx(pos_edge_index[0], PCH, 0)
    pb = _prep_idx(pos_edge_index[1], PCH, 0)
    na = _prep_idx(neg_edge_index[0], PCH, 0)
    nb = _prep_idx(neg_edge_index[1], PCH, 0)
    z1d = jnp.zeros((N_PAD,), jnp.float32)
    z2d = jnp.zeros((N_PAD, D_OUT), jnp.float32)

    d0, d1 = _deg_kernel(dst, z1d)
    ya, yb, dinv = _enc1(x_p, W1, d0.reshape(N_PAD, 1), d1.reshape(N_PAD, 1))
    pa0, pa1 = _scatter_kernel(ya, src, dst, z2d)
    yb_seq, _ = lax.optimization_barrier((yb, pa0))
    pb0, pb1 = _scatter_kernel(yb_seq, src, dst, z2d)
    y2 = _enc2(pa0, pa1, pb0, pb1, ya, yb, dinv, b1.reshape(1, H), W2)
    q0, q1 = _scatter_kernel(y2, src, dst, z2d)
    z = _dec(q0, q1, y2, dinv, b2.reshape(1, D_OUT))
    ps, ns = _decode_kernel(z, pa, pb, na, nb)
    return ps[:P], ns[:P]


# speedup vs baseline: 1.3567x; 1.3567x over previous
import functools

import jax
import jax.numpy as jnp
from jax import lax
from jax.experimental import pallas as pl
from jax.experimental.pallas import tpu as pltpu
from jax.experimental.pallas import tpu_sc as plsc

N = 10000
E = 320000
P = 160000
D_IN = 128
H = 128
D_OUT = 64

NC = 2
NS = 16
NW = NC * NS
L = 16

N_PAD = 10240
ROWS_PT = N_PAD // NS
ECW = 64
ECH = 160
EW_PAD = ECH * ECW
PCH = 40
PW_PAD = PCH * 128
SC_NBUF = 5
SC_LOOK = 4
DC_NBUF = 4
DC_LOOK = 3

_mesh = plsc.VectorSubcoreMesh(core_axis_name="c", subcore_axis_name="s")
_sc_params = pltpu.CompilerParams(use_tc_tiling_on_sc=False)
_sc_params_nl = pltpu.CompilerParams(use_tc_tiling_on_sc=False,
                                     needs_layout_passes=False)


def _wid():
    return lax.axis_index("c") * NS + lax.axis_index("s")


@functools.partial(
    pl.kernel,
    out_type=(jax.ShapeDtypeStruct((N_PAD,), jnp.float32),
              jax.ShapeDtypeStruct((N_PAD,), jnp.float32)),
    mesh=_mesh,
    scratch_types=[
        pltpu.VMEM((ECH, ECW), jnp.int32),
        pltpu.VMEM((ECW,), jnp.float32),
        pltpu.VMEM_SHARED((N_PAD,), jnp.float32),
    ],
    compiler_params=_sc_params,
)
def _deg_kernel(dst_hbm, zeros_hbm, out0, out1, dstv, onesv, acc):
    c = lax.axis_index("c")
    s = lax.axis_index("s")
    wid = _wid()
    base = s * ROWS_PT
    pltpu.sync_copy(zeros_hbm.at[pl.ds(base, ROWS_PT)],
                    acc.at[pl.ds(base, ROWS_PT)])
    pltpu.sync_copy(dst_hbm.at[wid], dstv)
    ones16 = jnp.ones((L,), jnp.float32)
    for i in range(ECW // L):
        onesv[pl.ds(i * L, L)] = ones16
    plsc.subcore_barrier()

    def body(j, carry):
        pltpu.sync_copy(onesv, acc.at[dstv.at[j]], add=True)
        return carry

    lax.fori_loop(0, ECH, body, 0)
    plsc.subcore_barrier()

    @pl.when(c == 0)
    def _():
        pltpu.sync_copy(acc.at[pl.ds(base, ROWS_PT)],
                        out0.at[pl.ds(base, ROWS_PT)])

    @pl.when(c == 1)
    def _():
        pltpu.sync_copy(acc.at[pl.ds(base, ROWS_PT)],
                        out1.at[pl.ds(base, ROWS_PT)])


@functools.partial(
    pl.kernel,
    out_type=(jax.ShapeDtypeStruct((N_PAD, D_OUT), jnp.float32),
              jax.ShapeDtypeStruct((N_PAD, D_OUT), jnp.float32)),
    mesh=_mesh,
    scratch_types=(
        [pltpu.VMEM((ECH, ECW), jnp.int32),
         pltpu.VMEM((ECH, ECW), jnp.int32)]
        + [pltpu.VMEM((ECW, D_OUT), jnp.float32)] * SC_NBUF
        + [pltpu.VMEM_SHARED((N_PAD, D_OUT), jnp.float32)]
        + [pltpu.VMEM_SHARED((N_PAD, D_OUT), jnp.float32)]
        + [pltpu.SemaphoreType.DMA] * (2 * SC_NBUF)
    ),
    compiler_params=_sc_params,
)
def _scatter_kernel(y_hbm, src_hbm, dst_hbm, zeros_hbm, out0, out1,
                    *rest):
    srcv, dstv = rest[0], rest[1]
    rowsv = list(rest[2:2 + SC_NBUF])
    acc = rest[2 + SC_NBUF]
    ys = rest[3 + SC_NBUF]
    sems = rest[4 + SC_NBUF:]
    sem_g = sems[:SC_NBUF]
    sem_s = sems[SC_NBUF:]
    c = lax.axis_index("c")
    s = lax.axis_index("s")
    wid = _wid()
    base = s * ROWS_PT
    pltpu.sync_copy(src_hbm.at[wid], srcv)
    pltpu.sync_copy(dst_hbm.at[wid], dstv)
    pltpu.sync_copy(y_hbm.at[pl.ds(base, ROWS_PT)],
                    ys.at[pl.ds(base, ROWS_PT)])
    pltpu.sync_copy(zeros_hbm.at[pl.ds(base, ROWS_PT)],
                    acc.at[pl.ds(base, ROWS_PT)])
    plsc.subcore_barrier()
    for t in range(SC_LOOK):
        pltpu.async_copy(ys.at[srcv.at[t]], rowsv[t % SC_NBUF],
                         sem_g[t % SC_NBUF])

    def outer(jo, carry):
        t0 = jo * SC_NBUF
        for u in range(SC_NBUF):
            t = t0 + u
            bn = (u + SC_LOOK) % SC_NBUF
            tn = t + SC_LOOK
            pltpu.make_async_copy(ys.at[srcv.at[u]], rowsv[u],
                                  sem_g[u]).wait()

            @pl.when(tn >= SC_NBUF)
            def _():
                pltpu.make_async_copy(rowsv[bn], acc.at[dstv.at[u]],
                                      sem_s[bn]).wait()

            @pl.when(tn < ECH)
            def _():
                pltpu.async_copy(ys.at[srcv.at[tn]], rowsv[bn],
                                 sem_g[bn])

            pltpu.async_copy(rowsv[u], acc.at[dstv.at[t]], sem_s[u],
                             add=True)
        return carry

    lax.fori_loop(0, ECH // SC_NBUF, outer, 0)
    for t in range(ECH - (SC_NBUF - SC_LOOK), ECH):
        b = t % SC_NBUF
        pltpu.make_async_copy(rowsv[b], acc.at[dstv.at[b]], sem_s[b]).wait()
    plsc.subcore_barrier()

    @pl.when(c == 0)
    def _():
        pltpu.sync_copy(acc.at[pl.ds(base, ROWS_PT)],
                        out0.at[pl.ds(base, ROWS_PT)])

    @pl.when(c == 1)
    def _():
        pltpu.sync_copy(acc.at[pl.ds(base, ROWS_PT)],
                        out1.at[pl.ds(base, ROWS_PT)])


@functools.partial(
    pl.kernel,
    out_type=(jax.ShapeDtypeStruct((NW * PW_PAD,), jnp.float32),
              jax.ShapeDtypeStruct((NW * PW_PAD,), jnp.float32)),
    mesh=_mesh,
    scratch_types=(
        [pltpu.VMEM((PCH, 128), jnp.int32),
         pltpu.VMEM((PCH, 128), jnp.int32)]
        + [pltpu.VMEM((128, D_OUT), jnp.float32)] * (2 * DC_NBUF)
        + [pltpu.VMEM((PW_PAD,), jnp.float32)]
        + [pltpu.VMEM_SHARED((N_PAD, D_OUT), jnp.float32)]
        + [pltpu.SemaphoreType.DMA] * (2 * DC_NBUF)
    ),
    compiler_params=_sc_params_nl,
)
def _decode_kernel(z_hbm, pa_hbm, pb_hbm, na_hbm, nb_hbm, outp, outn,
                   *rest):
    av, bv = rest[0], rest[1]
    za = list(rest[2:2 + DC_NBUF])
    zb = list(rest[2 + DC_NBUF:2 + 2 * DC_NBUF])
    scv = rest[2 + 2 * DC_NBUF]
    zs = rest[3 + 2 * DC_NBUF]
    sems = rest[4 + 2 * DC_NBUF:]
    sem_a = sems[:DC_NBUF]
    sem_b = sems[DC_NBUF:]
    wid = _wid()
    iota = lax.iota(jnp.int32, L)
    base = lax.axis_index("s") * ROWS_PT
    pltpu.sync_copy(z_hbm.at[pl.ds(base, ROWS_PT)],
                    zs.at[pl.ds(base, ROWS_PT)])
    plsc.subcore_barrier()

    def run(a_hbm, b_hbm, out_hbm):
        pltpu.sync_copy(a_hbm.at[wid], av)
        pltpu.sync_copy(b_hbm.at[wid], bv)
        for t in range(DC_LOOK):
            b = t % DC_NBUF
            pltpu.async_copy(zs.at[av.at[t]], za[b], sem_a[b])
            pltpu.async_copy(zs.at[bv.at[t]], zb[b], sem_b[b])

        def outer(jo, carry):
            t0 = jo * DC_NBUF
            for u in range(DC_NBUF):
                t = t0 + u
                bn = (u + DC_LOOK) % DC_NBUF
                tn = t + DC_LOOK
                pltpu.make_async_copy(zs.at[av.at[u]], za[u],
                                      sem_a[u]).wait()
                pltpu.make_async_copy(zs.at[bv.at[u]], zb[u],
                                      sem_b[u]).wait()

                @pl.when(tn < PCH)
                def _():
                    pltpu.async_copy(zs.at[av.at[tn]], za[bn], sem_a[bn])
                    pltpu.async_copy(zs.at[bv.at[tn]], zb[bn], sem_b[bn])

                def group(g, carry2):
                    rows = g * L + iota
                    accs = [jnp.zeros((L,), jnp.float32) for _ in range(2)]
                    for dcol in range(D_OUT):
                        col = jnp.full((L,), dcol, jnp.int32)
                        accs[dcol % 2] = accs[dcol % 2] + (
                            plsc.load_gather(za[u], [rows, col]) *
                            plsc.load_gather(zb[u], [rows, col]))
                    acc = accs[0] + accs[1]
                    scv[pl.ds(t * 128 + g * L, L)] = acc
                    return carry2

                lax.fori_loop(0, 128 // L, group, 0)
            return carry

        lax.fori_loop(0, PCH // DC_NBUF, outer, 0)
        pltpu.sync_copy(scv, out_hbm.at[pl.ds(wid * PW_PAD, PW_PAD)])

    run(pa_hbm, pb_hbm, outp)
    run(na_hbm, nb_hbm, outn)


def _enc1_body(x_ref, w1_ref, d0_ref, d1_ref, ya_ref, yb_ref, dinv_ref):
    deg = d0_ref[...] + d1_ref[...] + 1.0
    dinv = lax.rsqrt(deg)
    dinv_ref[...] = dinv
    xw = jnp.dot(x_ref[...], w1_ref[...], preferred_element_type=jnp.float32)
    y = xw * dinv
    ya_ref[...] = y[:, :D_OUT]
    yb_ref[...] = y[:, D_OUT:]


_enc1 = pl.pallas_call(
    _enc1_body,
    out_shape=(jax.ShapeDtypeStruct((N_PAD, D_OUT), jnp.float32),
               jax.ShapeDtypeStruct((N_PAD, D_OUT), jnp.float32),
               jax.ShapeDtypeStruct((N_PAD, 1), jnp.float32)),
)


def _enc2_body(pa0_ref, pa1_ref, pb0_ref, pb1_ref, ya_ref, yb_ref,
               dinv_ref, b1_ref, w2_ref, y2_ref):
    ha = pa0_ref[...] + pa1_ref[...] + ya_ref[...]
    hb = pb0_ref[...] + pb1_ref[...] + yb_ref[...]
    h = jnp.concatenate([ha, hb], axis=1) * dinv_ref[...] + b1_ref[...]
    valid = (lax.broadcasted_iota(jnp.int32, (N_PAD, 1), 0) < N)
    z1 = jnp.where(valid, jnp.maximum(h, 0.0), 0.0)
    xw2 = jnp.dot(z1, w2_ref[...], preferred_element_type=jnp.float32)
    y2_ref[...] = xw2 * dinv_ref[...]


_enc2 = pl.pallas_call(
    _enc2_body,
    out_shape=jax.ShapeDtypeStruct((N_PAD, D_OUT), jnp.float32),
)


def _dec_body(q0_ref, q1_ref, y2_ref, dinv_ref, b2_ref, z_ref):
    z = (q0_ref[...] + q1_ref[...] + y2_ref[...]) * dinv_ref[...] + b2_ref[...]
    valid = (lax.broadcasted_iota(jnp.int32, (N_PAD, 1), 0) < N)
    z_ref[...] = jnp.where(valid, z, 0.0)


_dec = pl.pallas_call(
    _dec_body,
    out_shape=jax.ShapeDtypeStruct((N_PAD, D_OUT), jnp.float32),
)


def _prep_idx(row, nch, width, fill):
    pad = NW * nch * width - row.shape[0]
    arr = jnp.concatenate([row, jnp.full((pad,), fill, jnp.int32)])
    return arr.reshape(NW, nch, width)


def kernel(x, edge_index, pos_edge_index, neg_edge_index, W1, b1, W2, b2):
    x_p = jnp.pad(x, ((0, N_PAD - N), (0, 0)))
    src = _prep_idx(edge_index[0], ECH, ECW, N_PAD - 1)
    dst = _prep_idx(edge_index[1], ECH, ECW, N_PAD - 1)
    pa = _prep_idx(pos_edge_index[0], PCH, 128, 0)
    pb = _prep_idx(pos_edge_index[1], PCH, 128, 0)
    na = _prep_idx(neg_edge_index[0], PCH, 128, 0)
    nb = _prep_idx(neg_edge_index[1], PCH, 128, 0)
    z1d = jnp.zeros((N_PAD,), jnp.float32)
    z2d = jnp.zeros((N_PAD, D_OUT), jnp.float32)

    d0, d1 = _deg_kernel(dst, z1d)
    ya, yb, dinv = _enc1(x_p, W1, d0.reshape(N_PAD, 1), d1.reshape(N_PAD, 1))
    pa0, pa1 = _scatter_kernel(ya, src, dst, z2d)
    yb_seq, _ = lax.optimization_barrier((yb, pa0))
    pb0, pb1 = _scatter_kernel(yb_seq, src, dst, z2d)
    y2 = _enc2(pa0, pa1, pb0, pb1, ya, yb, dinv, b1.reshape(1, H), W2)
    q0, q1 = _scatter_kernel(y2, src, dst, z2d)
    z = _dec(q0, q1, y2, dinv, b2.reshape(1, D_OUT))
    ps, ns = _decode_kernel(z, pa, pb, na, nb)
    return ps[:P], ns[:P]


# speedup vs baseline: 2.4317x; 1.7923x over previous
import functools

import jax
import jax.numpy as jnp
from jax import lax
from jax.experimental import pallas as pl
from jax.experimental.pallas import tpu as pltpu
from jax.experimental.pallas import tpu_sc as plsc

N = 10000
E = 320000
P = 160000
D_IN = 128
H = 128
D_OUT = 64

NC = 2
NS = 16
NW = NC * NS
L = 16

N_PAD = 10240
ROWS_PT = N_PAD // NS
ECW = 64
ECH = 160
EW_PAD = ECH * ECW
PCH = 40
PW_PAD = PCH * 128
SC_NBUF = 5
SC_LOOK = 4
DC_NBUF = 4
DC_LOOK = 3

_mesh = plsc.VectorSubcoreMesh(core_axis_name="c", subcore_axis_name="s")
_sc_params = pltpu.CompilerParams(use_tc_tiling_on_sc=False)
_sc_params_nl = pltpu.CompilerParams(use_tc_tiling_on_sc=False,
                                     needs_layout_passes=False)


def _wid():
    return lax.axis_index("c") * NS + lax.axis_index("s")


@functools.partial(
    pl.kernel,
    out_type=(jax.ShapeDtypeStruct((N_PAD,), jnp.float32),
              jax.ShapeDtypeStruct((N_PAD,), jnp.float32)),
    mesh=_mesh,
    scratch_types=[
        pltpu.VMEM((ECH, ECW), jnp.int32),
        pltpu.VMEM((ECW,), jnp.float32),
        pltpu.VMEM_SHARED((N_PAD,), jnp.float32),
    ],
    compiler_params=_sc_params,
)
def _deg_kernel(dst_hbm, zeros_hbm, out0, out1, dstv, onesv, acc):
    c = lax.axis_index("c")
    s = lax.axis_index("s")
    wid = _wid()
    base = s * ROWS_PT
    pltpu.sync_copy(zeros_hbm.at[pl.ds(base, ROWS_PT)],
                    acc.at[pl.ds(base, ROWS_PT)])
    pltpu.sync_copy(dst_hbm.at[wid], dstv)
    ones16 = jnp.ones((L,), jnp.float32)
    for i in range(ECW // L):
        onesv[pl.ds(i * L, L)] = ones16
    plsc.subcore_barrier()

    def body(j, carry):
        pltpu.sync_copy(onesv, acc.at[dstv.at[j]], add=True)
        return carry

    lax.fori_loop(0, ECH, body, 0)
    plsc.subcore_barrier()

    @pl.when(c == 0)
    def _():
        pltpu.sync_copy(acc.at[pl.ds(base, ROWS_PT)],
                        out0.at[pl.ds(base, ROWS_PT)])

    @pl.when(c == 1)
    def _():
        pltpu.sync_copy(acc.at[pl.ds(base, ROWS_PT)],
                        out1.at[pl.ds(base, ROWS_PT)])


@functools.partial(
    pl.kernel,
    out_type=(jax.ShapeDtypeStruct((N_PAD, D_OUT), jnp.float32),
              jax.ShapeDtypeStruct((N_PAD, D_OUT), jnp.float32)),
    mesh=_mesh,
    scratch_types=(
        [pltpu.VMEM((ECH, ECW), jnp.int32),
         pltpu.VMEM((ECH, ECW), jnp.int32)]
        + [pltpu.VMEM((ECW, D_OUT), jnp.float32)] * SC_NBUF
        + [pltpu.VMEM_SHARED((N_PAD, D_OUT), jnp.float32)]
        + [pltpu.VMEM_SHARED((N_PAD, D_OUT), jnp.float32)]
        + [pltpu.SemaphoreType.DMA] * (2 * SC_NBUF)
    ),
    compiler_params=_sc_params,
)
def _scatter_kernel(y_hbm, src_hbm, dst_hbm, zeros_hbm, out0, out1,
                    *rest):
    srcv, dstv = rest[0], rest[1]
    rowsv = list(rest[2:2 + SC_NBUF])
    acc = rest[2 + SC_NBUF]
    ys = rest[3 + SC_NBUF]
    sems = rest[4 + SC_NBUF:]
    sem_g = sems[:SC_NBUF]
    sem_s = sems[SC_NBUF:]
    c = lax.axis_index("c")
    s = lax.axis_index("s")
    wid = _wid()
    base = s * ROWS_PT
    pltpu.sync_copy(src_hbm.at[wid], srcv)
    pltpu.sync_copy(dst_hbm.at[wid], dstv)
    pltpu.sync_copy(y_hbm.at[pl.ds(base, ROWS_PT)],
                    ys.at[pl.ds(base, ROWS_PT)])
    pltpu.sync_copy(zeros_hbm.at[pl.ds(base, ROWS_PT)],
                    acc.at[pl.ds(base, ROWS_PT)])
    plsc.subcore_barrier()
    for t in range(SC_LOOK):
        pltpu.async_copy(ys.at[srcv.at[t]], rowsv[t % SC_NBUF],
                         sem_g[t % SC_NBUF])

    def outer(jo, carry):
        t0 = jo * SC_NBUF
        for u in range(SC_NBUF):
            t = t0 + u
            bn = (u + SC_LOOK) % SC_NBUF
            tn = t + SC_LOOK
            pltpu.make_async_copy(ys.at[srcv.at[u]], rowsv[u],
                                  sem_g[u]).wait()

            @pl.when(tn >= SC_NBUF)
            def _():
                pltpu.make_async_copy(rowsv[bn], acc.at[dstv.at[u]],
                                      sem_s[bn]).wait()

            @pl.when(tn < ECH)
            def _():
                pltpu.async_copy(ys.at[srcv.at[tn]], rowsv[bn],
                                 sem_g[bn])

            pltpu.async_copy(rowsv[u], acc.at[dstv.at[t]], sem_s[u],
                             add=True)
        return carry

    lax.fori_loop(0, ECH // SC_NBUF, outer, 0)
    for t in range(ECH - (SC_NBUF - SC_LOOK), ECH):
        b = t % SC_NBUF
        pltpu.make_async_copy(rowsv[b], acc.at[dstv.at[b]], sem_s[b]).wait()
    plsc.subcore_barrier()

    @pl.when(c == 0)
    def _():
        pltpu.sync_copy(acc.at[pl.ds(base, ROWS_PT)],
                        out0.at[pl.ds(base, ROWS_PT)])

    @pl.when(c == 1)
    def _():
        pltpu.sync_copy(acc.at[pl.ds(base, ROWS_PT)],
                        out1.at[pl.ds(base, ROWS_PT)])


@functools.partial(
    pl.kernel,
    out_type=(jax.ShapeDtypeStruct((NW * PW_PAD,), jnp.float32),
              jax.ShapeDtypeStruct((NW * PW_PAD,), jnp.float32)),
    mesh=_mesh,
    scratch_types=(
        [pltpu.VMEM((PCH, 128), jnp.int32),
         pltpu.VMEM((PCH, 128), jnp.int32)]
        + [pltpu.VMEM((128, D_OUT), jnp.float32)] * (2 * DC_NBUF)
        + [pltpu.VMEM((PW_PAD,), jnp.float32)]
        + [pltpu.VMEM_SHARED((N_PAD, D_OUT), jnp.float32)]
        + [pltpu.SemaphoreType.DMA] * (2 * DC_NBUF)
    ),
    compiler_params=_sc_params_nl,
)
def _decode_kernel(z_hbm, pa_hbm, pb_hbm, na_hbm, nb_hbm, outp, outn,
                   *rest):
    av, bv = rest[0], rest[1]
    za = list(rest[2:2 + DC_NBUF])
    zb = list(rest[2 + DC_NBUF:2 + 2 * DC_NBUF])
    scv = rest[2 + 2 * DC_NBUF]
    zs = rest[3 + 2 * DC_NBUF]
    sems = rest[4 + 2 * DC_NBUF:]
    sem_a = sems[:DC_NBUF]
    sem_b = sems[DC_NBUF:]
    wid = _wid()
    iota = lax.iota(jnp.int32, L)
    m15 = iota == (L - 1)
    base = lax.axis_index("s") * ROWS_PT
    pltpu.sync_copy(z_hbm.at[pl.ds(base, ROWS_PT)],
                    zs.at[pl.ds(base, ROWS_PT)])
    plsc.subcore_barrier()

    def run(a_hbm, b_hbm, out_hbm):
        pltpu.sync_copy(a_hbm.at[wid], av)
        pltpu.sync_copy(b_hbm.at[wid], bv)
        for t in range(DC_LOOK):
            b = t % DC_NBUF
            pltpu.async_copy(zs.at[av.at[t]], za[b], sem_a[b])
            pltpu.async_copy(zs.at[bv.at[t]], zb[b], sem_b[b])

        def outer(jo, carry):
            t0 = jo * DC_NBUF
            for u in range(DC_NBUF):
                t = t0 + u
                bn = (u + DC_LOOK) % DC_NBUF
                tn = t + DC_LOOK
                pltpu.make_async_copy(zs.at[av.at[u]], za[u],
                                      sem_a[u]).wait()
                pltpu.make_async_copy(zs.at[bv.at[u]], zb[u],
                                      sem_b[u]).wait()

                @pl.when(tn < PCH)
                def _():
                    pltpu.async_copy(zs.at[av.at[tn]], za[bn], sem_a[bn])
                    pltpu.async_copy(zs.at[bv.at[tn]], zb[bn], sem_b[bn])

                def group4(g4, carry2):
                    e0 = g4 * 4
                    for k in range(4):
                        e = e0 + k
                        sacc = None
                        for cpos in range(D_OUT // L):
                            a_ = za[u][e, pl.ds(cpos * L, L)]
                            b_ = zb[u][e, pl.ds(cpos * L, L)]
                            prod = a_ * b_
                            sacc = prod if sacc is None else sacc + prod
                        cs = plsc.cumsum(sacc)
                        idxv = jnp.full((L,), t * 128 + e, jnp.int32)
                        plsc.store_scatter(scv, [idxv], cs, mask=m15)
                    return carry2

                lax.fori_loop(0, 128 // 4, group4, 0)
            return carry

        lax.fori_loop(0, PCH // DC_NBUF, outer, 0)
        pltpu.sync_copy(scv, out_hbm.at[pl.ds(wid * PW_PAD, PW_PAD)])

    run(pa_hbm, pb_hbm, outp)
    run(na_hbm, nb_hbm, outn)


def _enc1_body(x_ref, w1_ref, d0_ref, d1_ref, ya_ref, yb_ref, dinv_ref):
    deg = d0_ref[...] + d1_ref[...] + 1.0
    dinv = lax.rsqrt(deg)
    dinv_ref[...] = dinv
    xw = jnp.dot(x_ref[...], w1_ref[...], preferred_element_type=jnp.float32)
    y = xw * dinv
    ya_ref[...] = y[:, :D_OUT]
    yb_ref[...] = y[:, D_OUT:]


_enc1 = pl.pallas_call(
    _enc1_body,
    out_shape=(jax.ShapeDtypeStruct((N_PAD, D_OUT), jnp.float32),
               jax.ShapeDtypeStruct((N_PAD, D_OUT), jnp.float32),
               jax.ShapeDtypeStruct((N_PAD, 1), jnp.float32)),
)


def _enc2_body(pa0_ref, pa1_ref, pb0_ref, pb1_ref, ya_ref, yb_ref,
               dinv_ref, b1_ref, w2_ref, y2_ref):
    ha = pa0_ref[...] + pa1_ref[...] + ya_ref[...]
    hb = pb0_ref[...] + pb1_ref[...] + yb_ref[...]
    h = jnp.concatenate([ha, hb], axis=1) * dinv_ref[...] + b1_ref[...]
    valid = (lax.broadcasted_iota(jnp.int32, (N_PAD, 1), 0) < N)
    z1 = jnp.where(valid, jnp.maximum(h, 0.0), 0.0)
    xw2 = jnp.dot(z1, w2_ref[...], preferred_element_type=jnp.float32)
    y2_ref[...] = xw2 * dinv_ref[...]


_enc2 = pl.pallas_call(
    _enc2_body,
    out_shape=jax.ShapeDtypeStruct((N_PAD, D_OUT), jnp.float32),
)


def _dec_body(q0_ref, q1_ref, y2_ref, dinv_ref, b2_ref, z_ref):
    z = (q0_ref[...] + q1_ref[...] + y2_ref[...]) * dinv_ref[...] + b2_ref[...]
    valid = (lax.broadcasted_iota(jnp.int32, (N_PAD, 1), 0) < N)
    z_ref[...] = jnp.where(valid, z, 0.0)


_dec = pl.pallas_call(
    _dec_body,
    out_shape=jax.ShapeDtypeStruct((N_PAD, D_OUT), jnp.float32),
)


def _prep_idx(row, nch, width, fill):
    pad = NW * nch * width - row.shape[0]
    arr = jnp.concatenate([row, jnp.full((pad,), fill, jnp.int32)])
    return arr.reshape(NW, nch, width)


def kernel(x, edge_index, pos_edge_index, neg_edge_index, W1, b1, W2, b2):
    x_p = jnp.pad(x, ((0, N_PAD - N), (0, 0)))
    src = _prep_idx(edge_index[0], ECH, ECW, N_PAD - 1)
    dst = _prep_idx(edge_index[1], ECH, ECW, N_PAD - 1)
    pa = _prep_idx(pos_edge_index[0], PCH, 128, 0)
    pb = _prep_idx(pos_edge_index[1], PCH, 128, 0)
    na = _prep_idx(neg_edge_index[0], PCH, 128, 0)
    nb = _prep_idx(neg_edge_index[1], PCH, 128, 0)
    z1d = jnp.zeros((N_PAD,), jnp.float32)
    z2d = jnp.zeros((N_PAD, D_OUT), jnp.float32)

    d0, d1 = _deg_kernel(dst, z1d)
    ya, yb, dinv = _enc1(x_p, W1, d0.reshape(N_PAD, 1), d1.reshape(N_PAD, 1))
    pa0, pa1 = _scatter_kernel(ya, src, dst, z2d)
    yb_seq, _ = lax.optimization_barrier((yb, pa0))
    pb0, pb1 = _scatter_kernel(yb_seq, src, dst, z2d)
    y2 = _enc2(pa0, pa1, pb0, pb1, ya, yb, dinv, b1.reshape(1, H), W2)
    q0, q1 = _scatter_kernel(y2, src, dst, z2d)
    z = _dec(q0, q1, y2, dinv, b2.reshape(1, D_OUT))
    ps, ns = _decode_kernel(z, pa, pb, na, nb)
    return ps[:P], ns[:P]
